# initial kernel scaffold (unmeasured)
import jax
import jax.numpy as jnp
from jax import lax
from jax.experimental import pallas as pl
from jax.experimental.pallas import tpu as pltpu

N_DEV = 4
MB = 2048
NH = 2048
NT = 1024
GELU_C = 0.7978845608028654


def _gelu(y):
    return 0.5 * y * (1.0 + jnp.tanh(GELU_C * (y + 0.044715 * y * y * y)))


def kernel(x, w_mat):
    m, ks = x.shape
    _, n = w_mat.shape
    assert m == N_DEV * MB and n == 2 * NH

    def body(x_hbm, w_ref, out_hbm, comm, xb, stage,
             send_sems, recv_sems, load_sem, store_sem):
        my = lax.axis_index("i")
        left = lax.rem(my + N_DEV - 1, N_DEV)
        right = lax.rem(my + 1, N_DEV)

        barrier = pltpu.get_barrier_semaphore()
        for nbr in (left, right):
            pl.semaphore_signal(
                barrier, inc=1,
                device_id=(nbr,), device_id_type=pl.DeviceIdType.MESH,
            )
        pl.semaphore_wait(barrier, 2)

        def load_x(c):
            cp = pltpu.make_async_copy(
                x_hbm.at[pl.ds(c * MB, MB), :], xb, load_sem)
            cp.start()
            cp.wait()

        for p in range(2):
            n0 = p * NH
            for h in range(N_DEV - 1):
                g = p * (N_DEV - 1) + h
                s = g % 2
                r = (g + 1) % 2
                c = lax.rem(my + 2 * N_DEV - 1 - h, N_DEV)
                load_x(c)
                for t in range(NH // NT):
                    col = n0 + t * NT
                    part = jnp.dot(
                        xb[...], w_ref[:, col:col + NT],
                        preferred_element_type=jnp.float32)
                    if h == 0:
                        comm[s, :, t * NT:(t + 1) * NT] = part.astype(jnp.bfloat16)
                    else:
                        prev = comm[s, :, t * NT:(t + 1) * NT].astype(jnp.float32)
                        comm[s, :, t * NT:(t + 1) * NT] = (prev + part).astype(jnp.bfloat16)
                rdma = pltpu.make_async_remote_copy(
                    src_ref=comm.at[s],
                    dst_ref=comm.at[r],
                    send_sem=send_sems.at[s],
                    recv_sem=recv_sems.at[r],
                    device_id=(right,),
                    device_id_type=pl.DeviceIdType.MESH,
                )
                rdma.start()
                rdma.wait()

            rf = (p * (N_DEV - 1) + N_DEV - 1) % 2
            load_x(my)
            for t in range(NH // NT):
                col = n0 + t * NT
                part = jnp.dot(
                    xb[...], w_ref[:, col:col + NT],
                    preferred_element_type=jnp.float32)
                acc = comm[rf, :, t * NT:(t + 1) * NT].astype(jnp.float32) + part
                stage[...] = _gelu(acc)
                cp = pltpu.make_async_copy(
                    stage, out_hbm.at[:, col:col + NT], store_sem)
                cp.start()
                cp.wait()

    return pl.pallas_call(
        body,
        out_shape=jax.ShapeDtypeStruct((MB, n), jnp.float32),
        in_specs=[
            pl.BlockSpec(memory_space=pl.ANY),
            pl.BlockSpec(memory_space=pltpu.VMEM),
        ],
        out_specs=pl.BlockSpec(memory_space=pl.ANY),
        scratch_shapes=[
            pltpu.VMEM((2, MB, NH), jnp.bfloat16),
            pltpu.VMEM((MB, ks), jnp.bfloat16),
            pltpu.VMEM((MB, NT), jnp.float32),
            pltpu.SemaphoreType.DMA((2,)),
            pltpu.SemaphoreType.DMA((2,)),
            pltpu.SemaphoreType.DMA,
            pltpu.SemaphoreType.DMA,
        ],
        compiler_params=pltpu.CompilerParams(collective_id=0),
    )(x, w_mat)


# baseline (device time: 839225 ns/iter reference)
import jax
import jax.numpy as jnp
from jax import lax
from jax.experimental import pallas as pl
from jax.experimental.pallas import tpu as pltpu

N_DEV = 4
MB = 2048
NH = 2048
NT = 512
GELU_C = 0.7978845608028654


def _gelu(y):
    return 0.5 * y * (1.0 + jnp.tanh(GELU_C * (y + 0.044715 * y * y * y)))


def kernel(x, w_mat):
    m, ks = x.shape
    _, n = w_mat.shape
    assert m == N_DEV * MB and n == 2 * NH
    x = x.astype(jnp.bfloat16)
    w_mat = w_mat.astype(jnp.bfloat16)

    def body(x_hbm, w_ref, out_hbm, comm, xb, stage,
             send_sems, recv_sems, load_sem, store_sem):
        my = lax.axis_index("i")
        left = lax.rem(my + N_DEV - 1, N_DEV)
        right = lax.rem(my + 1, N_DEV)

        barrier = pltpu.get_barrier_semaphore()
        for nbr in (left, right):
            pl.semaphore_signal(
                barrier, inc=1,
                device_id=(nbr,), device_id_type=pl.DeviceIdType.MESH,
            )
        pl.semaphore_wait(barrier, 2)

        def load_x(c):
            cp = pltpu.make_async_copy(
                x_hbm.at[pl.ds(c * MB, MB), :], xb, load_sem)
            cp.start()
            cp.wait()

        for p in range(2):
            n0 = p * NH
            for h in range(N_DEV - 1):
                g = p * (N_DEV - 1) + h
                s = g % 2
                r = (g + 1) % 2
                c = lax.rem(my + 2 * N_DEV - 1 - h, N_DEV)
                load_x(c)
                for t in range(NH // NT):
                    col = n0 + t * NT
                    part = jnp.dot(
                        xb[...], w_ref[:, col:col + NT],
                        preferred_element_type=jnp.float32)
                    if h == 0:
                        comm[s, :, t * NT:(t + 1) * NT] = part.astype(jnp.bfloat16)
                    else:
                        prev = comm[s, :, t * NT:(t + 1) * NT].astype(jnp.float32)
                        comm[s, :, t * NT:(t + 1) * NT] = (prev + part).astype(jnp.bfloat16)
                rdma = pltpu.make_async_remote_copy(
                    src_ref=comm.at[s],
                    dst_ref=comm.at[r],
                    send_sem=send_sems.at[s],
                    recv_sem=recv_sems.at[r],
                    device_id=(right,),
                    device_id_type=pl.DeviceIdType.MESH,
                )
                rdma.start()
                rdma.wait()

            rf = (p * (N_DEV - 1) + N_DEV - 1) % 2
            load_x(my)
            for t in range(NH // NT):
                col = n0 + t * NT
                part = jnp.dot(
                    xb[...], w_ref[:, col:col + NT],
                    preferred_element_type=jnp.float32)
                acc = comm[rf, :, t * NT:(t + 1) * NT].astype(jnp.float32) + part
                stage[...] = _gelu(acc)
                cp = pltpu.make_async_copy(
                    stage, out_hbm.at[:, col:col + NT], store_sem)
                cp.start()
                cp.wait()

    return pl.pallas_call(
        body,
        out_shape=jax.ShapeDtypeStruct((MB, n), jnp.float32),
        in_specs=[
            pl.BlockSpec(memory_space=pl.ANY),
            pl.BlockSpec(memory_space=pltpu.VMEM),
        ],
        out_specs=pl.BlockSpec(memory_space=pl.ANY),
        scratch_shapes=[
            pltpu.VMEM((2, MB, NH), jnp.bfloat16),
            pltpu.VMEM((MB, ks), jnp.bfloat16),
            pltpu.VMEM((MB, NT), jnp.float32),
            pltpu.SemaphoreType.DMA((2,)),
            pltpu.SemaphoreType.DMA((2,)),
            pltpu.SemaphoreType.DMA,
            pltpu.SemaphoreType.DMA,
        ],
        compiler_params=pltpu.CompilerParams(
            collective_id=0,
            vmem_limit_bytes=64 * 1024 * 1024,
        ),
    )(x, w_mat)
